# Initial kernel scaffold; baseline (speedup 1.0000x reference)
import jax
import jax.numpy as jnp
from jax.experimental import pallas as pl


def kernel(users, items, features, user_neighbors, kg_neighbors, W_self1, W_neigh1, b1, W_self2, W_neigh2, b2):
    raise NotImplementedError("write your pallas kernel here")


# speedup vs baseline: 2.9037x; 2.9037x over previous
import functools

import jax
import jax.numpy as jnp
from jax import lax
from jax.experimental import pallas as pl
from jax.experimental.pallas import tpu as pltpu
from jax.experimental.pallas import tpu_sc as plsc

NS = 10
D = 128
HID = 128
OUT = 64
BLK = 128
CHUNK = 128
NWORKERS = 32


def _sc_gather_rows(table, idx, n_pad):
    cols = table.shape[1]
    mesh = plsc.VectorSubcoreMesh(core_axis_name="c", subcore_axis_name="s")
    idx2 = idx.reshape(1, n_pad)

    @functools.partial(
        pl.kernel,
        out_type=jax.ShapeDtypeStruct((n_pad, cols), table.dtype),
        mesh=mesh,
    )
    def gather_kernel(x_hbm, i_hbm, o_hbm):
        def body(i_vmem, o_vmem):
            pltpu.sync_copy(x_hbm.at[i_vmem.at[0]], o_vmem)

        pltpu.emit_pipeline(
            body,
            grid=(n_pad // CHUNK,),
            in_specs=[pl.BlockSpec((1, CHUNK), lambda i: (0, i))],
            out_specs=[pl.BlockSpec((CHUNK, cols), lambda i: (i, 0))],
            core_axis_name=("c", "s"),
            dimension_semantics=(pltpu.PARALLEL,),
        )(i_hbm, o_hbm)

    return gather_kernel(table, idx2)


def _make_tc_body(b, n_l0, nblk_l0, nblk_h1):
    inv_ns = 1.0 / NS

    def body(fs_ref, n0, n1, n2, n3, n4, n5, n6, n7, n8, n9,
             ws1_ref, wn1_ref, b1_ref, ws2_ref, wn2_ref, b2_ref,
             out_ref, acc_ref):
        i = pl.program_id(0)

        @pl.when(i == 0)
        def _():
            acc_ref[...] = jnp.zeros((n_l0, OUT), jnp.float32)

        fn = (n0[...] + n1[...] + n2[...] + n3[...] + n4[...] +
              n5[...] + n6[...] + n7[...] + n8[...] + n9[...]) * inv_ns
        h = jnp.maximum(
            jnp.dot(fs_ref[...], ws1_ref[...],
                    preferred_element_type=jnp.float32)
            + jnp.dot(fn, wn1_ref[...], preferred_element_type=jnp.float32)
            + b1_ref[...],
            0.0,
        )

        @pl.when(i < nblk_l0)
        def _():
            acc_ref[pl.ds(i * BLK, BLK), :] += jnp.dot(
                h, ws2_ref[...], preferred_element_type=jnp.float32)

        @pl.when(i >= nblk_l0)
        def _():
            j = lax.rem(i - nblk_l0, nblk_l0)
            acc_ref[pl.ds(j * BLK, BLK), :] += jnp.dot(
                h, wn2_ref[...], preferred_element_type=jnp.float32) * inv_ns

        @pl.when(i == nblk_h1 - 1)
        def _():
            emb = acc_ref[...] + b2_ref[...]
            nrm = jnp.maximum(
                jnp.sqrt(jnp.sum(emb * emb, axis=1, keepdims=True)), 1e-6)
            on = emb / nrm
            itm = on[0:b, :]
            rt = on[b:n_l0, :]
            p = lax.dot_general(rt, itm, (((1,), (1,)), ((), ())),
                                preferred_element_type=jnp.float32)
            rid = lax.broadcasted_iota(jnp.int32, (b * NS, b), 0)
            cid = lax.broadcasted_iota(jnp.int32, (b * NS, b), 1)
            sel = (rid >= cid * NS) & (rid < cid * NS + NS)
            out_ref[...] = jnp.sum(jnp.where(sel, p, 0.0), axis=0,
                                   keepdims=True)

    return body


def _tc_fused(all_rows, b, ws1, wn1, b1, ws2, wn2, b2):
    n_l0 = b * (1 + NS)
    n_h1 = n_l0 * (1 + NS)
    nblk_l0 = n_l0 // BLK
    nblk_h1 = n_h1 // BLK
    in_specs = [pl.BlockSpec((BLK, D), lambda i: (i, 0))]
    for k in range(NS):
        in_specs.append(
            pl.BlockSpec((BLK, D), lambda i, kk=k: (nblk_h1 * (1 + kk) + i, 0)))
    in_specs += [
        pl.BlockSpec((D, HID), lambda i: (0, 0)),
        pl.BlockSpec((D, HID), lambda i: (0, 0)),
        pl.BlockSpec((1, HID), lambda i: (0, 0)),
        pl.BlockSpec((HID, OUT), lambda i: (0, 0)),
        pl.BlockSpec((HID, OUT), lambda i: (0, 0)),
        pl.BlockSpec((1, OUT), lambda i: (0, 0)),
    ]
    return pl.pallas_call(
        _make_tc_body(b, n_l0, nblk_l0, nblk_h1),
        grid=(nblk_h1,),
        in_specs=in_specs,
        out_specs=pl.BlockSpec((1, b), lambda i: (0, 0)),
        out_shape=jax.ShapeDtypeStruct((1, b), jnp.float32),
        scratch_shapes=[pltpu.VMEM((n_l0, OUT), jnp.float32)],
    )(all_rows, *([all_rows] * NS), ws1, wn1, b1, ws2, wn2, b2)


def kernel(users, items, features, user_neighbors, kg_neighbors,
           W_self1, W_neigh1, b1, W_self2, W_neigh2, b2):
    b = users.shape[0]
    users = users.astype(jnp.int32)
    items = items.astype(jnp.int32)
    user_neighbors = user_neighbors.astype(jnp.int32)
    kg_neighbors = kg_neighbors.astype(jnp.int32)

    rated = jnp.take(user_neighbors, users, axis=0).reshape(-1)
    level0 = jnp.concatenate([items, rated])
    nbr0 = jnp.take(kg_neighbors, level0, axis=0)
    g = jnp.concatenate([level0, nbr0.T.reshape(-1)])
    nbg = jnp.take(kg_neighbors, g, axis=0)
    all_idx = jnp.concatenate([g, nbg.T.reshape(-1)])

    n_all = all_idx.shape[0]
    step = NWORKERS * CHUNK
    n_pad = ((n_all + step - 1) // step) * step
    all_idx = jnp.concatenate(
        [all_idx, jnp.zeros((n_pad - n_all,), jnp.int32)])

    all_rows = _sc_gather_rows(features, all_idx, n_pad)
    pred = _tc_fused(all_rows, b, W_self1, W_neigh1, b1.reshape(1, HID),
                     W_self2, W_neigh2, b2.reshape(1, OUT))
    return pred.reshape(b)


# speedup vs baseline: 3.6808x; 1.2676x over previous
import functools

import jax
import jax.numpy as jnp
from jax import lax
from jax.experimental import pallas as pl
from jax.experimental.pallas import tpu as pltpu
from jax.experimental.pallas import tpu_sc as plsc

NS = 10
D = 128
HID = 128
OUT = 64
BLK = 512
CHUNK = 128
NWORKERS = 32


def _sc_gather_rows(table, idx, n_pad):
    cols = table.shape[1]
    mesh = plsc.VectorSubcoreMesh(core_axis_name="c", subcore_axis_name="s")
    idx2 = idx.reshape(1, n_pad)

    @functools.partial(
        pl.kernel,
        out_type=jax.ShapeDtypeStruct((n_pad, cols), table.dtype),
        mesh=mesh,
    )
    def gather_kernel(x_hbm, i_hbm, o_hbm):
        def body(i_vmem, o_vmem):
            pltpu.sync_copy(x_hbm.at[i_vmem.at[0]], o_vmem)

        pltpu.emit_pipeline(
            body,
            grid=(n_pad // CHUNK,),
            in_specs=[pl.BlockSpec((1, CHUNK), lambda i: (0, i))],
            out_specs=[pl.BlockSpec((CHUNK, cols), lambda i: (i, 0))],
            core_axis_name=("c", "s"),
            dimension_semantics=(pltpu.PARALLEL,),
        )(i_hbm, o_hbm)

    return gather_kernel(table, idx2)


def _make_tc_body(b, n_l0, nblk_l0, nblk_h1):
    inv_ns = 1.0 / NS

    def body(fs_ref, n0, n1, n2, n3, n4, n5, n6, n7, n8, n9,
             ws1_ref, wn1_ref, b1_ref, ws2_ref, wn2_ref, b2_ref,
             out_ref, acc_ref):
        i = pl.program_id(0)

        @pl.when(i == 0)
        def _():
            acc_ref[...] = jnp.zeros((n_l0, OUT), jnp.float32)

        fn = (n0[...] + n1[...] + n2[...] + n3[...] + n4[...] +
              n5[...] + n6[...] + n7[...] + n8[...] + n9[...]) * inv_ns
        h = jnp.maximum(
            jnp.dot(fs_ref[...], ws1_ref[...],
                    preferred_element_type=jnp.float32)
            + jnp.dot(fn, wn1_ref[...], preferred_element_type=jnp.float32)
            + b1_ref[...],
            0.0,
        )

        @pl.when(i < nblk_l0)
        def _():
            acc_ref[pl.ds(i * BLK, BLK), :] += jnp.dot(
                h, ws2_ref[...], preferred_element_type=jnp.float32)

        @pl.when(i >= nblk_l0)
        def _():
            j = lax.rem(i - nblk_l0, nblk_l0)
            acc_ref[pl.ds(j * BLK, BLK), :] += jnp.dot(
                h, wn2_ref[...], preferred_element_type=jnp.float32) * inv_ns

        @pl.when(i == nblk_h1 - 1)
        def _():
            emb = acc_ref[...] + b2_ref[...]
            nrm = jnp.maximum(
                jnp.sqrt(jnp.sum(emb * emb, axis=1, keepdims=True)), 1e-6)
            on = emb / nrm
            itm = on[0:b, :]
            rt = on[b:n_l0, :]
            p = lax.dot_general(rt, itm, (((1,), (1,)), ((), ())),
                                preferred_element_type=jnp.float32)
            rid = lax.broadcasted_iota(jnp.int32, (b * NS, b), 0)
            cid = lax.broadcasted_iota(jnp.int32, (b * NS, b), 1)
            sel = (rid >= cid * NS) & (rid < cid * NS + NS)
            out_ref[...] = jnp.sum(jnp.where(sel, p, 0.0), axis=0,
                                   keepdims=True)

    return body


def _tc_fused(all_rows, b, ws1, wn1, b1, ws2, wn2, b2):
    n_l0 = b * (1 + NS)
    n_h1 = n_l0 * (1 + NS)
    nblk_l0 = n_l0 // BLK
    nblk_h1 = n_h1 // BLK
    in_specs = [pl.BlockSpec((BLK, D), lambda i: (i, 0))]
    for k in range(NS):
        in_specs.append(
            pl.BlockSpec((BLK, D), lambda i, kk=k: (nblk_h1 * (1 + kk) + i, 0)))
    in_specs += [
        pl.BlockSpec((D, HID), lambda i: (0, 0)),
        pl.BlockSpec((D, HID), lambda i: (0, 0)),
        pl.BlockSpec((1, HID), lambda i: (0, 0)),
        pl.BlockSpec((HID, OUT), lambda i: (0, 0)),
        pl.BlockSpec((HID, OUT), lambda i: (0, 0)),
        pl.BlockSpec((1, OUT), lambda i: (0, 0)),
    ]
    return pl.pallas_call(
        _make_tc_body(b, n_l0, nblk_l0, nblk_h1),
        grid=(nblk_h1,),
        in_specs=in_specs,
        out_specs=pl.BlockSpec((1, b), lambda i: (0, 0)),
        out_shape=jax.ShapeDtypeStruct((1, b), jnp.float32),
        scratch_shapes=[pltpu.VMEM((n_l0, OUT), jnp.float32)],
    )(all_rows, *([all_rows] * NS), ws1, wn1, b1, ws2, wn2, b2)


def kernel(users, items, features, user_neighbors, kg_neighbors,
           W_self1, W_neigh1, b1, W_self2, W_neigh2, b2):
    b = users.shape[0]
    users = users.astype(jnp.int32)
    items = items.astype(jnp.int32)
    user_neighbors = user_neighbors.astype(jnp.int32)
    kg_neighbors = kg_neighbors.astype(jnp.int32)

    rated = jnp.take(user_neighbors, users, axis=0).reshape(-1)
    level0 = jnp.concatenate([items, rated])
    nbr0 = jnp.take(kg_neighbors, level0, axis=0)
    g = jnp.concatenate([level0, nbr0.T.reshape(-1)])
    nbg = jnp.take(kg_neighbors, g, axis=0)
    all_idx = jnp.concatenate([g, nbg.T.reshape(-1)])

    n_all = all_idx.shape[0]
    step = NWORKERS * CHUNK
    n_pad = ((n_all + step - 1) // step) * step
    all_idx = jnp.concatenate(
        [all_idx, jnp.zeros((n_pad - n_all,), jnp.int32)])

    all_rows = _sc_gather_rows(features, all_idx, n_pad)
    pred = _tc_fused(all_rows, b, W_self1, W_neigh1, b1.reshape(1, HID),
                     W_self2, W_neigh2, b2.reshape(1, OUT))
    return pred.reshape(b)
